# baseline (speedup 1.0000x reference)
import functools

import jax
import jax.numpy as jnp
from jax import lax
from jax.experimental import pallas as pl
from jax.experimental.pallas import tpu as pltpu
from jax.experimental.pallas import tpu_sc as plsc

L = 16
HC = 32
NUM_CORES = 2
NUM_SUBCORES = 16
NUM_WORKERS = NUM_CORES * NUM_SUBCORES
EPSILON = 1.0


def _stats_body(x_ref, s_ref, m_ref, xt_ref):
    xb = x_ref[...]
    s_ref[...] = (jnp.max(jnp.abs(xb)) + 1e-6).reshape(1, 1)
    m_ref[...] = jnp.mean(xb, axis=1, keepdims=True)
    n_tiles = xb.shape[0] // HC
    xt_ref[...] = jnp.swapaxes(xb.reshape(n_tiles, HC, xb.shape[1]), 1, 2)


def _make_sc_kernel(batch, n_in, n_out, rows_per_w):
    n_halves = rows_per_w // HC
    mesh = plsc.VectorSubcoreMesh(
        core_axis_name="c", subcore_axis_name="s",
        num_cores=NUM_CORES, num_subcores=NUM_SUBCORES)

    @functools.partial(
        pl.kernel,
        out_type=jax.ShapeDtypeStruct((batch * n_out,), jnp.float32),
        mesh=mesh,
        scratch_types=[
            pltpu.VMEM((n_out * L,), jnp.int32),
            pltpu.VMEM((n_out * L,), jnp.float32),
            pltpu.VMEM((n_in * HC,), jnp.float32),
            pltpu.VMEM((n_out * HC,), jnp.float32),
            pltpu.VMEM((rows_per_w,), jnp.float32),
            pltpu.VMEM((L,), jnp.float32),
        ],
        compiler_params=pltpu.CompilerParams(needs_layout_passes=False),
    )
    def sc_kernel(xt_hbm, idx_hbm, w_hbm, mean_hbm, s_hbm, out_hbm,
                  idx_v, sw_v, xt_v, out_v, mean_v, s_v):
        wid = lax.axis_index("s") * NUM_CORES + lax.axis_index("c")
        row0 = wid * rows_per_w

        pltpu.sync_copy(idx_hbm, idx_v)
        pltpu.sync_copy(w_hbm, sw_v)
        pltpu.sync_copy(s_hbm, s_v)
        pltpu.sync_copy(mean_hbm.at[pl.ds(row0, rows_per_w)], mean_v)

        s_vec = s_v[...]
        inv_vec = 1.0 / s_vec
        s_scalar = s_vec[0]
        zeros = jnp.zeros((L,), jnp.float32)
        neginf = jnp.full((L,), -jnp.inf, jnp.float32)

        @plsc.parallel_loop(0, n_out, 1, unroll=4)
        def wmul(i):
            sw_v[pl.ds(i * L, L)] = sw_v[pl.ds(i * L, L)] * s_vec

        for h in range(n_halves):
            tile = (wid * n_halves + h)
            pltpu.sync_copy(xt_hbm.at[pl.ds(tile * n_in * HC, n_in * HC)],
                            xt_v)

            mb0 = mean_v[pl.ds(h * HC, L)]
            mb1 = mean_v[pl.ds(h * HC + L, L)]

            @plsc.parallel_loop(0, n_in, 1, unroll=4, carry=(zeros, zeros))
            def enc(i, gm):
                gm0, gm1 = gm
                x0 = xt_v[pl.ds(i * HC, L)]
                x1 = xt_v[pl.ds(i * HC + L, L)]
                a0 = jnp.abs(x0 - mb0)
                a1 = jnp.abs(x1 - mb1)
                xt_v[pl.ds(i * HC, L)] = jnp.where(a0 < s_vec, x0, neginf)
                xt_v[pl.ds(i * HC + L, L)] = jnp.where(a1 < s_vec, x1,
                                                       neginf)
                return (jnp.maximum(gm0, a0), jnp.maximum(gm1, a1))

            gm0, gm1 = enc
            any_closed = (lax.reduce_max(jnp.maximum(gm0, gm1), axes=(0,))
                          >= s_scalar)

            def fast_half():
                @plsc.parallel_loop(0, n_out, 1, unroll=2)
                def o_body(o):
                    ivb = idx_v[pl.ds(o * L, L)] * HC
                    wv = sw_v[pl.ds(o * L, L)]
                    for sub in range(2):
                        accs = [None] * 4
                        for k in range(L):
                            g = xt_v[pl.ds(ivb[k] + sub * L, L)]
                            v = g + jnp.full((L,), wv[k], jnp.float32)
                            a = accs[k % 4]
                            accs[k % 4] = (v if a is None
                                           else jnp.maximum(a, v))
                        acc = jnp.maximum(jnp.maximum(accs[0], accs[1]),
                                          jnp.maximum(accs[2], accs[3]))
                        out_v[pl.ds(o * HC + sub * L, L)] = acc * inv_vec

            def slow_half():
                @plsc.parallel_loop(0, n_out, 1, unroll=2)
                def o_body(o):
                    ivb = idx_v[pl.ds(o * L, L)] * HC
                    wv = sw_v[pl.ds(o * L, L)]
                    for sub in range(2):
                        accs = [None] * 4
                        for k in range(L):
                            g = xt_v[pl.ds(ivb[k] + sub * L, L)]
                            v = g + jnp.full((L,), wv[k], jnp.float32)
                            v = jnp.where(g == neginf, zeros, v)
                            a = accs[k % 4]
                            accs[k % 4] = (v if a is None
                                           else jnp.maximum(a, v))
                        acc = jnp.maximum(jnp.maximum(accs[0], accs[1]),
                                          jnp.maximum(accs[2], accs[3]))
                        out_v[pl.ds(o * HC + sub * L, L)] = acc * inv_vec

            lax.cond(any_closed, slow_half, fast_half)
            pltpu.sync_copy(out_v,
                            out_hbm.at[pl.ds(tile * n_out * HC, n_out * HC)])

    return sc_kernel


def kernel(x, weights, src_idx):
    batch, n_in = x.shape
    n_out = src_idx.shape[0] // L
    rows_per_w = batch // NUM_WORKERS
    n_halves = rows_per_w // HC

    s11, m2d, xt = pl.pallas_call(
        _stats_body,
        out_shape=[
            jax.ShapeDtypeStruct((1, 1), jnp.float32),
            jax.ShapeDtypeStruct((batch, 1), jnp.float32),
            jax.ShapeDtypeStruct((NUM_WORKERS * n_halves, n_in, HC),
                                 jnp.float32),
        ],
    )(x)

    s_vec = jnp.broadcast_to(s11[0, 0], (L,))
    means = m2d.reshape(-1)
    xt = xt.reshape(-1)

    out = _make_sc_kernel(batch, n_in, n_out, rows_per_w)(
        xt, src_idx, weights, means, s_vec)

    out = out.reshape(NUM_WORKERS * n_halves, n_out, HC)
    return jnp.swapaxes(out, 1, 2).reshape(batch, n_out)


# speedup vs baseline: 1.6202x; 1.6202x over previous
import functools

import jax
import jax.numpy as jnp
from jax import lax
from jax.experimental import pallas as pl
from jax.experimental.pallas import tpu as pltpu
from jax.experimental.pallas import tpu_sc as plsc

L = 16
NUM_CORES = 2
NUM_SUBCORES = 16
NUM_WORKERS = NUM_CORES * NUM_SUBCORES
EPSILON = 1.0


def _stats_body(x_ref, s_ref, m_ref):
    xb = x_ref[...]
    s_ref[...] = (jnp.max(jnp.abs(xb)) + 1e-6).reshape(1, 1)
    m_ref[...] = jnp.broadcast_to(jnp.mean(xb, axis=1, keepdims=True),
                                  (xb.shape[0], L))


def _make_sc_kernel(batch, n_in, n_out, rows_per_w, ch):
    n_groups = n_out // L
    mesh = plsc.VectorSubcoreMesh(
        core_axis_name="c", subcore_axis_name="s",
        num_cores=NUM_CORES, num_subcores=NUM_SUBCORES)

    @functools.partial(
        pl.kernel,
        out_type=jax.ShapeDtypeStruct((batch * n_out,), jnp.float32),
        mesh=mesh,
        scratch_types=[
            pltpu.VMEM((n_out * L,), jnp.int32),
            pltpu.VMEM((n_out * L,), jnp.float32),
            pltpu.VMEM((ch * n_in,), jnp.float32),
            pltpu.VMEM((ch // 2 * n_in,), jnp.int32),
            pltpu.VMEM((ch * n_out,), jnp.float32),
            pltpu.VMEM((rows_per_w * L,), jnp.float32),
            pltpu.VMEM((L,), jnp.float32),
        ],
        compiler_params=pltpu.CompilerParams(needs_layout_passes=False),
    )
    def sc_kernel(x_hbm, idx_hbm, w_hbm, mean_hbm, s_hbm, out_hbm,
                  idx_v, sw_v, x_v, y_v, out_v, mean_v, s_v):
        wid = lax.axis_index("s") * NUM_CORES + lax.axis_index("c")
        row0 = wid * rows_per_w
        npair = ch // 2

        pltpu.sync_copy(idx_hbm, idx_v)
        pltpu.sync_copy(w_hbm, sw_v)
        pltpu.sync_copy(s_hbm, s_v)
        pltpu.sync_copy(mean_hbm.at[pl.ds(row0 * L, rows_per_w * L)], mean_v)

        s_vec = s_v[...]
        inv_vec = 1.0 / s_vec
        s_scalar = s_vec[0]
        zeros = jnp.zeros((L,), jnp.float32)
        zeros2 = jnp.zeros((2 * L,), jnp.bfloat16)
        neginf = jnp.full((L,), -jnp.inf, jnp.float32)
        neginf2 = jnp.full((2 * L,), -jnp.inf, jnp.bfloat16)

        @plsc.parallel_loop(0, n_out, 1, unroll=4)
        def wmul(i):
            sv = sw_v[pl.ds(i * L, L)] * s_vec
            swp = plsc.pack(sv, sv, format=plsc.PackFormat.INTERLEAVED)
            sw_v[pl.ds(i * L, L)] = plsc.bitcast(swp, jnp.float32)

        for c in range(rows_per_w // ch):
            base = row0 + c * ch
            pltpu.sync_copy(x_hbm.at[pl.ds(base * n_in, ch * n_in)], x_v)

            def enc_pair(p, gm_pair):
                mb0 = mean_v[pl.ds((c * ch + 2 * p) * L, L)]
                mb1 = mean_v[pl.ds((c * ch + 2 * p + 1) * L, L)]

                @plsc.parallel_loop(0, n_in // L, 1, unroll=4,
                                    carry=gm_pair)
                def enc_i(i, gm):
                    x0 = x_v[pl.ds(2 * p * n_in + i * L, L)]
                    x1 = x_v[pl.ds((2 * p + 1) * n_in + i * L, L)]
                    a0 = jnp.abs(x0 - mb0)
                    a1 = jnp.abs(x1 - mb1)
                    y0 = jnp.where(a0 < s_vec, x0, neginf)
                    y1 = jnp.where(a1 < s_vec, x1, neginf)
                    yp = plsc.pack(y0, y1,
                                   format=plsc.PackFormat.INTERLEAVED)
                    y_v[pl.ds(p * n_in + i * L, L)] = plsc.bitcast(
                        yp, jnp.int32)
                    return jnp.maximum(gm, jnp.maximum(a0, a1))

                return enc_i

            gmax = lax.fori_loop(0, npair, enc_pair, zeros)
            any_closed = lax.reduce_max(gmax, axes=(0,)) >= s_scalar

            def make_chunk_fn(slow):
                def chunk_fn():
                    def og_body(og, _):
                        ivs = [idx_v[pl.ds(og * (L * L) + k * L, L)]
                               for k in range(L)]
                        sws = [plsc.bitcast(
                                   sw_v[pl.ds(og * (L * L) + k * L, L)],
                                   jnp.bfloat16)
                               for k in range(L)]

                        @plsc.parallel_loop(0, npair, 1, unroll=2)
                        def p_body(p):
                            row = y_v.at[pl.ds(p * n_in, n_in)]
                            accs = [None] * 4
                            for k in range(L):
                                gi = plsc.load_gather(row, [ivs[k]])
                                gb = plsc.bitcast(gi, jnp.bfloat16)
                                v = gb + sws[k]
                                if slow:
                                    v = jnp.where(gb == neginf2, zeros2, v)
                                a = accs[k % 4]
                                accs[k % 4] = (v if a is None
                                               else jnp.maximum(a, v))
                            acc = jnp.maximum(
                                jnp.maximum(accs[0], accs[1]),
                                jnp.maximum(accs[2], accs[3]))
                            r0, r1 = plsc.unpack(
                                acc, format=plsc.PackFormat.INTERLEAVED)
                            o0 = 2 * p * n_out + og * L
                            out_v[pl.ds(o0, L)] = r0 * inv_vec
                            out_v[pl.ds(o0 + n_out, L)] = r1 * inv_vec

                        return 0

                    lax.fori_loop(0, n_groups, og_body, 0)
                return chunk_fn

            lax.cond(any_closed, make_chunk_fn(True), make_chunk_fn(False))
            pltpu.sync_copy(out_v, out_hbm.at[pl.ds(base * n_out, ch * n_out)])

    return sc_kernel


def kernel(x, weights, src_idx):
    batch, n_in = x.shape
    n_out = src_idx.shape[0] // L
    rows_per_w = batch // NUM_WORKERS
    ch = min(rows_per_w, 32)

    s11, m2d = pl.pallas_call(
        _stats_body,
        out_shape=[
            jax.ShapeDtypeStruct((1, 1), jnp.float32),
            jax.ShapeDtypeStruct((batch, L), jnp.float32),
        ],
    )(x)

    s_vec = jnp.broadcast_to(s11[0, 0], (L,))
    means = m2d.reshape(-1)

    idx2 = src_idx.reshape(n_out, L)
    w2 = weights.reshape(n_out, L)
    lane = jnp.arange(n_out, dtype=jnp.int32)[:, None] % L
    order = jnp.argsort(jnp.bitwise_and(idx2 - lane, L - 1), axis=1)
    idx2 = jnp.take_along_axis(idx2, order, axis=1)
    w2 = jnp.take_along_axis(w2, order, axis=1)

    idx_t = idx2.reshape(n_out // L, L, L).transpose(0, 2, 1).reshape(-1)
    w_t = w2.reshape(n_out // L, L, L).transpose(0, 2, 1).reshape(-1)

    sc = _make_sc_kernel(batch, n_in, n_out, rows_per_w, ch)
    return sc(x.reshape(-1), idx_t, w_t, means, s_vec).reshape(batch, n_out)


# speedup vs baseline: 1.8968x; 1.1707x over previous
import functools

import jax
import jax.numpy as jnp
from jax import lax
from jax.experimental import pallas as pl
from jax.experimental.pallas import tpu as pltpu
from jax.experimental.pallas import tpu_sc as plsc

L = 16
NUM_CORES = 2
NUM_SUBCORES = 16
NUM_WORKERS = NUM_CORES * NUM_SUBCORES
EPSILON = 1.0


def _stats_body(x_ref, s_ref, m_ref):
    xb = x_ref[...]
    s_ref[...] = (jnp.max(jnp.abs(xb)) + 1e-6).reshape(1, 1)
    m_ref[...] = jnp.broadcast_to(jnp.mean(xb, axis=1, keepdims=True),
                                  (xb.shape[0], L))


def _make_sc_kernel(batch, n_in, n_out, rows_per_w, ch):
    n_groups = n_out // L
    mesh = plsc.VectorSubcoreMesh(
        core_axis_name="c", subcore_axis_name="s",
        num_cores=NUM_CORES, num_subcores=NUM_SUBCORES)

    @functools.partial(
        pl.kernel,
        out_type=jax.ShapeDtypeStruct((batch * n_out,), jnp.float32),
        mesh=mesh,
        scratch_types=[
            pltpu.VMEM((n_out * L,), jnp.int32),
            pltpu.VMEM((n_out * L,), jnp.float32),
            pltpu.VMEM((ch * n_in,), jnp.float32),
            pltpu.VMEM((ch // 2 * n_in,), jnp.int32),
            pltpu.VMEM((ch * n_out,), jnp.float32),
            pltpu.VMEM((rows_per_w * L,), jnp.float32),
            pltpu.VMEM((L,), jnp.float32),
        ],
        compiler_params=pltpu.CompilerParams(needs_layout_passes=False),
    )
    def sc_kernel(x_hbm, idx_hbm, w_hbm, mean_hbm, s_hbm, out_hbm,
                  idx_v, sw_v, x_v, y_v, out_v, mean_v, s_v):
        wid = lax.axis_index("s") * NUM_CORES + lax.axis_index("c")
        row0 = wid * rows_per_w
        npair = ch // 2

        pltpu.sync_copy(idx_hbm, idx_v)
        pltpu.sync_copy(w_hbm, sw_v)
        pltpu.sync_copy(s_hbm, s_v)
        pltpu.sync_copy(mean_hbm.at[pl.ds(row0 * L, rows_per_w * L)], mean_v)

        s_vec = s_v[...]
        inv_vec = 1.0 / s_vec
        s_scalar = s_vec[0]
        zeros = jnp.zeros((L,), jnp.float32)
        zeros2 = jnp.zeros((2 * L,), jnp.bfloat16)
        neginf = jnp.full((L,), -jnp.inf, jnp.float32)
        neginf2 = jnp.full((2 * L,), -jnp.inf, jnp.bfloat16)

        @plsc.parallel_loop(0, n_out, 1, unroll=4)
        def wmul(i):
            sv = sw_v[pl.ds(i * L, L)] * s_vec
            swp = plsc.pack(sv, sv, format=plsc.PackFormat.INTERLEAVED)
            sw_v[pl.ds(i * L, L)] = plsc.bitcast(swp, jnp.float32)

        for c in range(rows_per_w // ch):
            base = row0 + c * ch
            pltpu.sync_copy(x_hbm.at[pl.ds(base * n_in, ch * n_in)], x_v)

            def enc_pair(p, gm_pair):
                mb0 = mean_v[pl.ds((c * ch + 2 * p) * L, L)]
                mb1 = mean_v[pl.ds((c * ch + 2 * p + 1) * L, L)]

                @plsc.parallel_loop(0, n_in // L, 1, unroll=4,
                                    carry=gm_pair)
                def enc_i(i, gm):
                    x0 = x_v[pl.ds(2 * p * n_in + i * L, L)]
                    x1 = x_v[pl.ds((2 * p + 1) * n_in + i * L, L)]
                    a0 = jnp.abs(x0 - mb0)
                    a1 = jnp.abs(x1 - mb1)
                    y0 = jnp.where(a0 < s_vec, x0, neginf)
                    y1 = jnp.where(a1 < s_vec, x1, neginf)
                    yp = plsc.pack(y0, y1,
                                   format=plsc.PackFormat.INTERLEAVED)
                    y_v[pl.ds(p * n_in + i * L, L)] = plsc.bitcast(
                        yp, jnp.int32)
                    return jnp.maximum(gm, jnp.maximum(a0, a1))

                return enc_i

            gmax = lax.fori_loop(0, npair, enc_pair, zeros)
            any_closed = lax.reduce_max(gmax, axes=(0,)) >= s_scalar

            def make_chunk_fn(slow):
                def chunk_fn():
                    def og_body(og, _):
                        ivs = [idx_v[pl.ds(og * (L * L) + k * L, L)]
                               for k in range(L)]
                        sws = [plsc.bitcast(
                                   sw_v[pl.ds(og * (L * L) + k * L, L)],
                                   jnp.bfloat16)
                               for k in range(L)]

                        @plsc.parallel_loop(0, npair, 1, unroll=2)
                        def p_body(p):
                            row = y_v.at[pl.ds(p * n_in, n_in)]
                            accs = [None] * 4
                            for k in range(L):
                                gi = plsc.load_gather(row, [ivs[k]])
                                gb = plsc.bitcast(gi, jnp.bfloat16)
                                v = gb + sws[k]
                                if slow:
                                    v = jnp.where(gb == neginf2, zeros2, v)
                                a = accs[k % 4]
                                accs[k % 4] = (v if a is None
                                               else jnp.maximum(a, v))
                            acc = jnp.maximum(
                                jnp.maximum(accs[0], accs[1]),
                                jnp.maximum(accs[2], accs[3]))
                            r0, r1 = plsc.unpack(
                                acc, format=plsc.PackFormat.INTERLEAVED)
                            o0 = 2 * p * n_out + og * L
                            out_v[pl.ds(o0, L)] = r0 * inv_vec
                            out_v[pl.ds(o0 + n_out, L)] = r1 * inv_vec

                        return 0

                    lax.fori_loop(0, n_groups, og_body, 0)
                return chunk_fn

            lax.cond(any_closed, make_chunk_fn(True), make_chunk_fn(False))
            pltpu.sync_copy(out_v, out_hbm.at[pl.ds(base * n_out, ch * n_out)])

    return sc_kernel


def kernel(x, weights, src_idx):
    batch, n_in = x.shape
    n_out = src_idx.shape[0] // L
    rows_per_w = batch // NUM_WORKERS
    ch = min(rows_per_w, 32)

    s11, m2d = pl.pallas_call(
        _stats_body,
        out_shape=[
            jax.ShapeDtypeStruct((1, 1), jnp.float32),
            jax.ShapeDtypeStruct((batch, L), jnp.float32),
        ],
    )(x)

    s_vec = jnp.broadcast_to(s11[0, 0], (L,))
    means = m2d.reshape(-1)

    idx2 = src_idx.reshape(n_out, L)
    w2 = weights.reshape(n_out, L)

    idx_t = idx2.reshape(n_out // L, L, L).transpose(0, 2, 1).reshape(-1)
    w_t = w2.reshape(n_out // L, L, L).transpose(0, 2, 1).reshape(-1)

    sc = _make_sc_kernel(batch, n_in, n_out, rows_per_w, ch)
    return sc(x.reshape(-1), idx_t, w_t, means, s_vec).reshape(batch, n_out)
